# baseline (speedup 1.0000x reference)
import functools

import jax
import jax.numpy as jnp
from jax import lax
from jax.experimental import pallas as pl
from jax.experimental.pallas import tpu as pltpu
from jax.experimental.pallas import tpu_sc as plsc

N_NODES = 10000
N_EDGES = 320000
D = 128
DW = D // 2
L = 16
NW = 32
EPW = N_EDGES // NW
CH = 80
NCH = -(-EPW // CH) + 1
EPC = NCH * CH
GPC = CH // L
CUNROLL = 5


def _normalize_rows(x):
    def body(x_ref, o_ref):
        xb = x_ref[...]
        n2 = jnp.sum(xb * xb, axis=1, keepdims=True)
        xn = xb * lax.rsqrt(jnp.maximum(n2, 1e-16))
        lo = lax.bitcast_convert_type(
            xn[:, :DW].astype(jnp.bfloat16), jnp.uint16).astype(jnp.uint32)
        hi = lax.bitcast_convert_type(
            xn[:, DW:].astype(jnp.bfloat16), jnp.uint16).astype(jnp.uint32)
        o_ref[...] = lax.bitcast_convert_type((hi << 16) | lo, jnp.int32)

    return pl.pallas_call(
        body, out_shape=jax.ShapeDtypeStruct((N_NODES, DW), jnp.int32)
    )(x)


@functools.partial(
    pl.kernel,
    out_type=jax.ShapeDtypeStruct((NW, 2, L), jnp.float32),
    mesh=plsc.VectorSubcoreMesh(core_axis_name="c", subcore_axis_name="s"),
    compiler_params=pltpu.CompilerParams(
        needs_layout_passes=False, use_tc_tiling_on_sc=False),
    scratch_types=[
        pltpu.VMEM((EPW,), jnp.int32),
        pltpu.VMEM((EPW,), jnp.int32),
        pltpu.VMEM((EPW,), jnp.float32),
        pltpu.VMEM((EPC,), jnp.int32),
        pltpu.VMEM((EPC,), jnp.int32),
        pltpu.VMEM((CH, DW), jnp.int32),
        pltpu.VMEM((CH, DW), jnp.int32),
        pltpu.VMEM((CH, DW), jnp.int32),
        pltpu.VMEM((CH, DW), jnp.int32),
        pltpu.VMEM((CH, DW), jnp.int32),
        pltpu.VMEM((CH, DW), jnp.int32),
        pltpu.VMEM((L,), jnp.float32),
        pltpu.VMEM((L, L), jnp.float32),
        pltpu.VMEM((2, L), jnp.float32),
        pltpu.SemaphoreType.DMA,
        pltpu.SemaphoreType.DMA,
        pltpu.SemaphoreType.DMA,
    ],
)
def _edge_loss_sc(src_hbm, dst_hbm, w_hbm, xn_hbm, thrd_hbm, out_hbm,
                  idx_s, idx_d, w_v, idx_sc, idx_dc,
                  rs0, rd0, rs1, rd1, rs2, rd2,
                  thrd_v, acc_scr, out_v, sem0, sem1, sem2):
    wid = lax.axis_index("s") * 2 + lax.axis_index("c")
    base = wid * EPW
    pltpu.sync_copy(thrd_hbm, thrd_v)
    pltpu.sync_copy(src_hbm.at[pl.ds(base, EPW)], idx_s)
    pltpu.sync_copy(dst_hbm.at[pl.ds(base, EPW)], idx_d)
    pltpu.sync_copy(w_hbm.at[pl.ds(base, EPW)], w_v)
    tv = thrd_v[...]
    zero = jnp.zeros((L,), jnp.float32)
    lanes = lax.iota(jnp.int32, L)
    izero = jnp.zeros((L,), jnp.int32)

    def zero_body(i, _):
        for u in range(CUNROLL):
            idx_sc[pl.ds((i * CUNROLL + u) * L, L)] = izero
            idx_dc[pl.ds((i * CUNROLL + u) * L, L)] = izero
        return 0

    lax.fori_loop(0, EPC // L // CUNROLL, zero_body, 0)

    def cmp_body(i, off_vec):
        for u in range(CUNROLL):
            g = i * CUNROLL + u
            w_vec = w_v[pl.ds(g * L, L)]
            m = w_vec > 0.0
            mi = m.astype(jnp.int32)
            pos = off_vec + plsc.cumsum(mi) - mi
            plsc.store_scatter(idx_sc, [pos], idx_s[pl.ds(g * L, L)], mask=m)
            plsc.store_scatter(idx_dc, [pos], idx_d[pl.ds(g * L, L)], mask=m)
            off_vec = off_vec + plsc.all_reduce_population_count(m)
        return off_vec

    n_vec = lax.fori_loop(0, EPW // L // CUNROLL, cmp_body, izero)
    n = n_vec[0]
    nch_live = (n + CH - 1) // CH

    bufs = [(rs0, rd0, sem0), (rs1, rd1, sem1), (rs2, rd2, sem2)]
    NBUF = len(bufs)

    H = CH // 2

    def issue(c, rs, rd, sem):
        pltpu.async_copy(
            xn_hbm.at[idx_sc.at[pl.ds(c * CH, H)]], rs.at[pl.ds(0, H)], sem)
        pltpu.async_copy(
            xn_hbm.at[idx_sc.at[pl.ds(c * CH + H, H)]],
            rs.at[pl.ds(H, H)], sem)
        pltpu.async_copy(
            xn_hbm.at[idx_dc.at[pl.ds(c * CH, H)]], rd.at[pl.ds(0, H)], sem)
        pltpu.async_copy(
            xn_hbm.at[idx_dc.at[pl.ds(c * CH + H, H)]],
            rd.at[pl.ds(H, H)], sem)

    def drain(rs, rd, sem):
        pltpu.make_async_copy(
            xn_hbm.at[pl.ds(0, H)], rs.at[pl.ds(0, H)], sem).wait()
        pltpu.make_async_copy(
            xn_hbm.at[pl.ds(0, H)], rs.at[pl.ds(H, H)], sem).wait()
        pltpu.make_async_copy(
            xn_hbm.at[pl.ds(0, H)], rd.at[pl.ds(0, H)], sem).wait()
        pltpu.make_async_copy(
            xn_hbm.at[pl.ds(0, H)], rd.at[pl.ds(H, H)], sem).wait()

    def compute(c, rows_s, rows_d, carry):
        loss_vec = carry
        for g in range(GPC):
            valid = (c * CH + g * L) + lanes < n_vec
            m_vec = jnp.where(valid, 1.0, 0.0).astype(jnp.float32)
            for e in range(L):
                r = g * L + e
                parts = []
                for k in range(DW // L):
                    a = plsc.bitcast(rows_s[r, pl.ds(k * L, L)], jnp.bfloat16)
                    b = plsc.bitcast(rows_d[r, pl.ds(k * L, L)], jnp.bfloat16)
                    plo, phi = plsc.unpack(
                        a * b, format=plsc.PackFormat.INTERLEAVED)
                    parts.append(plo)
                    parts.append(phi)
                while len(parts) > 1:
                    parts = [parts[i] + parts[i + 1]
                             for i in range(0, len(parts) - 1, 2)] + (
                                 [parts[-1]] if len(parts) % 2 else [])
                acc_scr[e, :] = parts[0]
            cols = [
                plsc.load_gather(acc_scr, [lanes, jnp.full((L,), j, jnp.int32)])
                for j in range(L)
            ]
            while len(cols) > 1:
                cols = [cols[i] + cols[i + 1] for i in range(0, len(cols), 2)]
            sims = cols[0]
            vals = jnp.maximum(tv - sims, 0.0) * m_vec
            loss_vec = loss_vec + vals
        return loss_vec

    for b in range(NBUF):
        @pl.when(b < nch_live)
        def _(b=b):
            issue(b, *bufs[b])

    def ring_body(i, carry):
        for p in range(NBUF):
            c = NBUF * i + p
            rs, rd, sem = bufs[p]

            def live(c=c, rs=rs, rd=rd, sem=sem, carry=carry):
                drain(rs, rd, sem)
                new = compute(c, rs, rd, carry)

                @pl.when(c + NBUF < nch_live)
                def _():
                    issue(c + NBUF, rs, rd, sem)

                return new

            carry = lax.cond(c < nch_live, live, lambda: carry)
        return carry

    loss_vec = lax.fori_loop(0, NCH // NBUF, ring_body, zero)

    out_v[0, :] = loss_vec
    out_v[1, :] = n_vec.astype(jnp.float32) * (1.0 / L)
    pltpu.sync_copy(out_v, out_hbm.at[wid])


def kernel(trigger_edge_index, trigger_edge_weights, x, thrd):
    xn = _normalize_rows(x)
    src = trigger_edge_index[0]
    dst = trigger_edge_index[1]
    thrd_vec = jnp.full((L,), thrd, jnp.float32)
    parts = _edge_loss_sc(src, dst, trigger_edge_weights, xn, thrd_vec)
    return jnp.sum(parts[:, 0, :]) / jnp.sum(parts[:, 1, :])


# speedup vs baseline: 1.0057x; 1.0057x over previous
import functools

import jax
import jax.numpy as jnp
from jax import lax
from jax.experimental import pallas as pl
from jax.experimental.pallas import tpu as pltpu
from jax.experimental.pallas import tpu_sc as plsc

N_NODES = 10000
N_EDGES = 320000
D = 128
DW = D // 2
L = 16
NW = 32
EPW = N_EDGES // NW
CH = 80
NCH = -(-EPW // CH)
EPC = NCH * CH
GPC = CH // L
CUNROLL = 5


def _normalize_rows(x):
    def body(x_ref, o_ref):
        xb = x_ref[...]
        n2 = jnp.sum(xb * xb, axis=1, keepdims=True)
        xn = xb * lax.rsqrt(jnp.maximum(n2, 1e-16))
        lo = lax.bitcast_convert_type(
            xn[:, :DW].astype(jnp.bfloat16), jnp.uint16).astype(jnp.uint32)
        hi = lax.bitcast_convert_type(
            xn[:, DW:].astype(jnp.bfloat16), jnp.uint16).astype(jnp.uint32)
        o_ref[...] = lax.bitcast_convert_type((hi << 16) | lo, jnp.int32)

    return pl.pallas_call(
        body, out_shape=jax.ShapeDtypeStruct((N_NODES, DW), jnp.int32)
    )(x)


@functools.partial(
    pl.kernel,
    out_type=jax.ShapeDtypeStruct((NW, 2, L), jnp.float32),
    mesh=plsc.VectorSubcoreMesh(core_axis_name="c", subcore_axis_name="s"),
    compiler_params=pltpu.CompilerParams(
        needs_layout_passes=False, use_tc_tiling_on_sc=False),
    scratch_types=[
        pltpu.VMEM((EPW,), jnp.int32),
        pltpu.VMEM((EPW,), jnp.int32),
        pltpu.VMEM((EPW,), jnp.float32),
        pltpu.VMEM((EPC,), jnp.int32),
        pltpu.VMEM((EPC,), jnp.int32),
        pltpu.VMEM((CH, DW), jnp.int32),
        pltpu.VMEM((CH, DW), jnp.int32),
        pltpu.VMEM((CH, DW), jnp.int32),
        pltpu.VMEM((CH, DW), jnp.int32),
        pltpu.VMEM((L,), jnp.float32),
        pltpu.VMEM((L, L), jnp.float32),
        pltpu.VMEM((2, L), jnp.float32),
        pltpu.SemaphoreType.DMA,
        pltpu.SemaphoreType.DMA,
    ],
)
def _edge_loss_sc(src_hbm, dst_hbm, w_hbm, xn_hbm, thrd_hbm, out_hbm,
                  idx_s, idx_d, w_v, idx_sc, idx_dc,
                  rs0, rd0, rs1, rd1,
                  thrd_v, acc_scr, out_v, sem0, sem1):
    wid = lax.axis_index("s") * 2 + lax.axis_index("c")
    base = wid * EPW
    pltpu.sync_copy(thrd_hbm, thrd_v)
    pltpu.sync_copy(src_hbm.at[pl.ds(base, EPW)], idx_s)
    pltpu.sync_copy(dst_hbm.at[pl.ds(base, EPW)], idx_d)
    pltpu.sync_copy(w_hbm.at[pl.ds(base, EPW)], w_v)
    tv = thrd_v[...]
    zero = jnp.zeros((L,), jnp.float32)
    lanes = lax.iota(jnp.int32, L)
    izero = jnp.zeros((L,), jnp.int32)

    def zero_body(i, _):
        for u in range(CUNROLL):
            idx_sc[pl.ds((i * CUNROLL + u) * L, L)] = izero
            idx_dc[pl.ds((i * CUNROLL + u) * L, L)] = izero
        return 0

    lax.fori_loop(0, EPC // L // CUNROLL, zero_body, 0)

    def cmp_body(i, off_vec):
        for u in range(CUNROLL):
            g = i * CUNROLL + u
            w_vec = w_v[pl.ds(g * L, L)]
            m = w_vec > 0.0
            mi = m.astype(jnp.int32)
            pos = off_vec + plsc.cumsum(mi) - mi
            plsc.store_scatter(idx_sc, [pos], idx_s[pl.ds(g * L, L)], mask=m)
            plsc.store_scatter(idx_dc, [pos], idx_d[pl.ds(g * L, L)], mask=m)
            off_vec = off_vec + plsc.all_reduce_population_count(m)
        return off_vec

    n_vec = lax.fori_loop(0, EPW // L // CUNROLL, cmp_body, izero)
    n = n_vec[0]
    nch_live = (n + CH - 1) // CH

    bufs = [(rs0, rd0, sem0), (rs1, rd1, sem1)]
    NBUF = len(bufs)

    H = CH // 2

    def issue(c, rs, rd, sem):
        pltpu.async_copy(
            xn_hbm.at[idx_sc.at[pl.ds(c * CH, H)]], rs.at[pl.ds(0, H)], sem)
        pltpu.async_copy(
            xn_hbm.at[idx_sc.at[pl.ds(c * CH + H, H)]],
            rs.at[pl.ds(H, H)], sem)
        pltpu.async_copy(
            xn_hbm.at[idx_dc.at[pl.ds(c * CH, H)]], rd.at[pl.ds(0, H)], sem)
        pltpu.async_copy(
            xn_hbm.at[idx_dc.at[pl.ds(c * CH + H, H)]],
            rd.at[pl.ds(H, H)], sem)

    def drain(rs, rd, sem):
        pltpu.make_async_copy(
            xn_hbm.at[pl.ds(0, H)], rs.at[pl.ds(0, H)], sem).wait()
        pltpu.make_async_copy(
            xn_hbm.at[pl.ds(0, H)], rs.at[pl.ds(H, H)], sem).wait()
        pltpu.make_async_copy(
            xn_hbm.at[pl.ds(0, H)], rd.at[pl.ds(0, H)], sem).wait()
        pltpu.make_async_copy(
            xn_hbm.at[pl.ds(0, H)], rd.at[pl.ds(H, H)], sem).wait()

    def compute(c, rows_s, rows_d, carry):
        loss_vec = carry
        for g in range(GPC):
            valid = (c * CH + g * L) + lanes < n_vec
            m_vec = jnp.where(valid, 1.0, 0.0).astype(jnp.float32)
            for e in range(L):
                r = g * L + e
                parts = []
                for k in range(DW // L):
                    a = plsc.bitcast(rows_s[r, pl.ds(k * L, L)], jnp.bfloat16)
                    b = plsc.bitcast(rows_d[r, pl.ds(k * L, L)], jnp.bfloat16)
                    plo, phi = plsc.unpack(
                        a * b, format=plsc.PackFormat.INTERLEAVED)
                    parts.append(plo)
                    parts.append(phi)
                while len(parts) > 1:
                    parts = [parts[i] + parts[i + 1]
                             for i in range(0, len(parts) - 1, 2)] + (
                                 [parts[-1]] if len(parts) % 2 else [])
                acc_scr[e, :] = parts[0]
            cols = [
                plsc.load_gather(acc_scr, [lanes, jnp.full((L,), j, jnp.int32)])
                for j in range(L)
            ]
            while len(cols) > 1:
                cols = [cols[i] + cols[i + 1] for i in range(0, len(cols), 2)]
            sims = cols[0]
            vals = jnp.maximum(tv - sims, 0.0) * m_vec
            loss_vec = loss_vec + vals
        return loss_vec

    for b in range(NBUF):
        @pl.when(b < nch_live)
        def _(b=b):
            issue(b, *bufs[b])

    def ring_body(i, carry):
        for p in range(NBUF):
            c = NBUF * i + p
            rs, rd, sem = bufs[p]

            def live(c=c, rs=rs, rd=rd, sem=sem, carry=carry):
                drain(rs, rd, sem)
                new = compute(c, rs, rd, carry)

                @pl.when(c + NBUF < nch_live)
                def _():
                    issue(c + NBUF, rs, rd, sem)

                return new

            carry = lax.cond(c < nch_live, live, lambda: carry)
        return carry

    carry = lax.fori_loop(0, NCH // NBUF, ring_body, zero)

    def live_last(carry=carry):
        drain(*bufs[0])
        return compute(NCH - 1, bufs[0][0], bufs[0][1], carry)

    loss_vec = lax.cond(NCH - 1 < nch_live, live_last, lambda: carry)

    out_v[0, :] = loss_vec
    out_v[1, :] = n_vec.astype(jnp.float32) * (1.0 / L)
    pltpu.sync_copy(out_v, out_hbm.at[wid])


def kernel(trigger_edge_index, trigger_edge_weights, x, thrd):
    xn = _normalize_rows(x)
    src = trigger_edge_index[0]
    dst = trigger_edge_index[1]
    thrd_vec = jnp.full((L,), thrd, jnp.float32)
    parts = _edge_loss_sc(src, dst, trigger_edge_weights, xn, thrd_vec)
    return jnp.sum(parts[:, 0, :]) / jnp.sum(parts[:, 1, :])
